# baseline (speedup 1.0000x reference)
import functools

import jax
import jax.numpy as jnp
from jax import lax
from jax.experimental import pallas as pl
from jax.experimental.pallas import tpu as pltpu
from jax.experimental.pallas import tpu_sc as plsc

VOCAB = 100000
EMBED = 128
BATCH = 16384
CTX = 6

NC = 2
NS = 16
L = 16
NW = NC * NS
BPW = BATCH // NW
CHUNK = 128
NCHUNK = BPW // CHUNK
NXB = 3
NSLOT = NCHUNK * CTX

_MESH = plsc.VectorSubcoreMesh(
    core_axis_name="c", subcore_axis_name="s", num_cores=NC, num_subcores=NS
)


@functools.partial(
    pl.kernel,
    out_type=jax.ShapeDtypeStruct((BATCH * CTX,), jnp.float32),
    mesh=_MESH,
    scratch_types=[
        pltpu.VMEM((NCHUNK, CHUNK), jnp.int32),
        [pltpu.VMEM((NCHUNK, CHUNK), jnp.int32) for _ in range(CTX)],
        [pltpu.VMEM((CHUNK, EMBED), jnp.float32) for _ in range(2)],
        [pltpu.VMEM((CHUNK, EMBED), jnp.float32) for _ in range(NXB)],
        pltpu.VMEM((CTX * CHUNK,), jnp.float32),
        [pltpu.VMEM((CHUNK * CTX,), jnp.float32) for _ in range(2)],
        [pltpu.SemaphoreType.DMA for _ in range(2)],
        [pltpu.SemaphoreType.DMA for _ in range(NXB)],
        [pltpu.SemaphoreType.DMA for _ in range(2)],
    ],
    compiler_params=pltpu.CompilerParams(needs_layout_passes=False),
)
def _w2v(center_hbm, ctxt_hbm, ctable_hbm, xtable_hbm, out_hbm,
         cidx, xidx, crows, xrows, dotv, outv, csems, xsems, osems):
    wid = lax.axis_index("s") * NC + lax.axis_index("c")
    lane = lax.iota(jnp.int32, L)

    stage = [pltpu.async_copy(center_hbm.at[wid, 0], cidx.at[0], osems[0])]
    for k in range(CTX):
        stage.append(pltpu.async_copy(ctxt_hbm.at[k, wid, 0], xidx[k].at[0],
                                      osems[0]))
    for cp in stage:
        cp.wait()

    def fire_center(g):
        return pltpu.async_copy(ctable_hbm.at[cidx.at[g]], crows[g % 2],
                                csems[g % 2])

    def fire_ctx(slot):
        g, k = divmod(slot, CTX)
        return pltpu.async_copy(xtable_hbm.at[xidx[k].at[g]],
                                xrows[slot % NXB], xsems[slot % NXB])

    pend_c = fire_center(0)
    pend_x = [fire_ctx(s) for s in range(NXB)]

    rest = pl.ds(1, NCHUNK - 1)
    stage = [pltpu.async_copy(center_hbm.at[wid, rest], cidx.at[rest],
                              osems[1])]
    for k in range(CTX):
        stage.append(pltpu.async_copy(ctxt_hbm.at[k, wid, rest],
                                      xidx[k].at[rest], osems[1]))
    for cp in stage:
        cp.wait()

    pend_o = [None, None]
    for g in range(NCHUNK):
        cb = g % 2
        pend_c.wait()
        if g + 1 < NCHUNK:
            pend_c = fire_center(g + 1)
        for k in range(CTX):
            slot = g * CTX + k
            xb = slot % NXB
            pend_x[xb].wait()

            def sub_body(sub, _, xb=xb, k=k, cb=cb):
                rows = lane + sub * L

                def dbody(d, acc):
                    dv = jnp.bitwise_and(lane + d, EMBED - 1)
                    c = plsc.load_gather(crows[cb], [rows, dv])
                    x = plsc.load_gather(xrows[xb], [rows, dv])
                    return acc + c * x

                acc = lax.fori_loop(0, EMBED, dbody,
                                    jnp.zeros((L,), jnp.float32))
                plsc.store_scatter(
                    dotv, [jnp.full((L,), k * CHUNK, jnp.int32) + rows], acc)
                return 0

            lax.fori_loop(0, CHUNK // L, sub_body, 0)
            if slot + NXB < NSLOT:
                pend_x[xb] = fire_ctx(slot + NXB)
        if pend_o[cb] is not None:
            pend_o[cb].wait()
        for sub in range(CHUNK // L):
            rows = lane + sub * L
            accs = [dotv[pl.ds(k * CHUNK + sub * L, L)] for k in range(CTX)]
            m = accs[0]
            for k in range(1, CTX):
                m = jnp.maximum(m, accs[k])
            es = [jnp.exp(a - m) for a in accs]
            tot = es[0]
            for k in range(1, CTX):
                tot = tot + es[k]
            inv = 1.0 / tot
            orow = rows * CTX
            for k in range(CTX):
                plsc.store_scatter(outv[cb], [orow + k], es[k] * inv)
        base = wid * BPW + g * CHUNK
        pend_o[cb] = pltpu.async_copy(
            outv[cb], out_hbm.at[pl.ds(base * CTX, CHUNK * CTX)], osems[cb])
    for cp in pend_o:
        if cp is not None:
            cp.wait()


def kernel(center, context, center_table, context_table):
    center_r = center.reshape(NW, NCHUNK, CHUNK)
    ctxt_r = context.T.reshape(CTX, NW, NCHUNK, CHUNK)
    out = _w2v(center_r, ctxt_r, center_table, context_table)
    return out.reshape(BATCH, CTX)


# speedup vs baseline: 2.0422x; 2.0422x over previous
import functools

import jax
import jax.numpy as jnp
from jax import lax
from jax.experimental import pallas as pl
from jax.experimental.pallas import tpu as pltpu
from jax.experimental.pallas import tpu_sc as plsc

VOCAB = 100000
EMBED = 128
BATCH = 16384
CTX = 6

NC = 2
NS = 16
L = 16
NW = NC * NS
BPW = BATCH // NW
CHUNK = 64
NCHUNK = BPW // CHUNK
NBUF = 2
NPAIR = CTX // 2

_MESH = plsc.VectorSubcoreMesh(
    core_axis_name="c", subcore_axis_name="s", num_cores=NC, num_subcores=NS
)


@functools.partial(
    pl.kernel,
    out_type=jax.ShapeDtypeStruct((BATCH * CTX,), jnp.float32),
    mesh=_MESH,
    scratch_types=[
        pltpu.VMEM((NCHUNK, CHUNK), jnp.int32),
        pltpu.VMEM((NCHUNK * NPAIR, 2 * CHUNK), jnp.int32),
        [pltpu.VMEM((CHUNK, EMBED), jnp.float32) for _ in range(NBUF)],
        [pltpu.VMEM((CTX * CHUNK, EMBED), jnp.float32) for _ in range(NBUF)],
        [pltpu.VMEM((CHUNK * CTX,), jnp.float32) for _ in range(NBUF)],
        [pltpu.SemaphoreType.DMA for _ in range(NBUF)],
        [pltpu.SemaphoreType.DMA for _ in range(NBUF)],
    ],
    compiler_params=pltpu.CompilerParams(needs_layout_passes=False),
)
def _w2v(center_hbm, ctxt_hbm, ctable_hbm, xtable_hbm, out_hbm,
         cidx, xidx, crows, xrows, outv, sems, osems):
    wid = lax.axis_index("s") * NC + lax.axis_index("c")
    stage = [
        pltpu.async_copy(center_hbm.at[wid, 0], cidx.at[0], osems[0]),
        pltpu.async_copy(ctxt_hbm.at[wid, pl.ds(0, 8)],
                         xidx.at[pl.ds(0, 8)], osems[0]),
    ]
    for cp in stage:
        cp.wait()

    def fire(g, buf):
        cps = [pltpu.async_copy(ctable_hbm.at[cidx.at[g]], crows[buf],
                                sems[buf])]
        for j in range(NPAIR):
            cps.append(pltpu.async_copy(
                xtable_hbm.at[xidx.at[g * NPAIR + j]],
                xrows[buf].at[pl.ds(j * 2 * CHUNK, 2 * CHUNK)],
                sems[buf]))
        return cps

    pend = fire(0, 0)
    nrest = NCHUNK * NPAIR - 8
    stage = [
        pltpu.async_copy(center_hbm.at[wid, pl.ds(1, NCHUNK - 1)],
                         cidx.at[pl.ds(1, NCHUNK - 1)], osems[1]),
        pltpu.async_copy(ctxt_hbm.at[wid, pl.ds(8, nrest)],
                         xidx.at[pl.ds(8, nrest)], osems[1]),
    ]
    for cp in stage:
        cp.wait()
    pend_out = [None] * NBUF
    for g in range(NCHUNK):
        buf = g % NBUF
        for cp in pend:
            cp.wait()
        if g + 1 < NCHUNK:
            pend = fire(g + 1, (g + 1) % NBUF)
        if pend_out[buf] is not None:
            pend_out[buf].wait()
        for sub in range(CHUNK // L):
            lane = lax.iota(jnp.int32, L)
            rows = lane + sub * L

            def dbody(d, accs):
                dv = jnp.bitwise_and(lane + d, EMBED - 1)
                c = plsc.load_gather(crows[buf], [rows, dv])
                return tuple(
                    accs[k]
                    + c * plsc.load_gather(xrows[buf], [rows + k * CHUNK, dv])
                    for k in range(CTX)
                )

            accs = lax.fori_loop(
                0, EMBED, dbody,
                tuple(jnp.zeros((L,), jnp.float32) for _ in range(CTX)),
            )
            m = accs[0]
            for k in range(1, CTX):
                m = jnp.maximum(m, accs[k])
            es = [jnp.exp(a - m) for a in accs]
            tot = es[0]
            for k in range(1, CTX):
                tot = tot + es[k]
            inv = 1.0 / tot
            orow = rows * CTX
            for k in range(CTX):
                plsc.store_scatter(outv[buf], [orow + k], es[k] * inv)
        base = wid * BPW + g * CHUNK
        pend_out[buf] = pltpu.async_copy(
            outv[buf], out_hbm.at[pl.ds(base * CTX, CHUNK * CTX)], osems[buf])
    for cp in pend_out:
        if cp is not None:
            cp.wait()


def kernel(center, context, center_table, context_table):
    center_r = center.reshape(NW, NCHUNK, CHUNK)
    ctxt_r = (context.T.reshape(CTX, NW, NCHUNK, CHUNK)
              .transpose(1, 2, 0, 3)
              .reshape(NW, NCHUNK * NPAIR, 2 * CHUNK))
    out = _w2v(center_r, ctxt_r, center_table, context_table)
    return out.reshape(BATCH, CTX)


# speedup vs baseline: 2.0639x; 1.0106x over previous
import functools

import jax
import jax.numpy as jnp
from jax import lax
from jax.experimental import pallas as pl
from jax.experimental.pallas import tpu as pltpu
from jax.experimental.pallas import tpu_sc as plsc

VOCAB = 100000
EMBED = 128
BATCH = 16384
CTX = 6

NC = 2
NS = 16
L = 16
NW = NC * NS
BPW = BATCH // NW
CHUNK = 64
NCHUNK = BPW // CHUNK
NBUF = 2

_MESH = plsc.VectorSubcoreMesh(
    core_axis_name="c", subcore_axis_name="s", num_cores=NC, num_subcores=NS
)


@functools.partial(
    pl.kernel,
    out_type=jax.ShapeDtypeStruct((BATCH * CTX,), jnp.float32),
    mesh=_MESH,
    scratch_types=[
        pltpu.VMEM((NCHUNK, CHUNK), jnp.int32),
        [pltpu.VMEM((NCHUNK, CHUNK), jnp.int32) for _ in range(CTX)],
        [pltpu.VMEM((CHUNK, EMBED), jnp.float32) for _ in range(NBUF)],
        [[pltpu.VMEM((CHUNK, EMBED), jnp.float32) for _ in range(CTX)]
         for _ in range(NBUF)],
        [pltpu.VMEM((CHUNK * CTX,), jnp.float32) for _ in range(NBUF)],
        [pltpu.SemaphoreType.DMA for _ in range(NBUF)],
        [pltpu.SemaphoreType.DMA for _ in range(NBUF)],
    ],
    compiler_params=pltpu.CompilerParams(needs_layout_passes=False),
)
def _w2v(center_hbm, ctxt_hbm, ctable_hbm, xtable_hbm, out_hbm,
         cidx, xidx, crows, xrows, outv, sems, osems):
    wid = lax.axis_index("s") * NC + lax.axis_index("c")
    stage = [pltpu.async_copy(center_hbm.at[wid, 0], cidx.at[0], osems[0])]
    for k in range(CTX):
        stage.append(pltpu.async_copy(ctxt_hbm.at[k, wid, 0], xidx[k].at[0],
                                      osems[0]))
    for cp in stage:
        cp.wait()

    def fire(g, buf):
        cps = [pltpu.async_copy(ctable_hbm.at[cidx.at[g]], crows[buf],
                                sems[buf])]
        for k in range(CTX):
            cps.append(pltpu.async_copy(xtable_hbm.at[xidx[k].at[g]],
                                        xrows[buf][k], sems[buf]))
        return cps

    pend = fire(0, 0)
    rest = pl.ds(1, NCHUNK - 1)
    stage = [pltpu.async_copy(center_hbm.at[wid, rest], cidx.at[rest],
                              osems[1])]
    for k in range(CTX):
        stage.append(pltpu.async_copy(ctxt_hbm.at[k, wid, rest],
                                      xidx[k].at[rest], osems[1]))
    for cp in stage:
        cp.wait()
    pend_out = [None] * NBUF
    for g in range(NCHUNK):
        buf = g % NBUF
        for cp in pend:
            cp.wait()
        if g + 1 < NCHUNK:
            pend = fire(g + 1, (g + 1) % NBUF)
        if pend_out[buf] is not None:
            pend_out[buf].wait()
        for sub in range(CHUNK // L):
            lane = lax.iota(jnp.int32, L)
            rows = lane + sub * L

            def dbody(d, accs):
                dv = jnp.bitwise_and(lane + d, EMBED - 1)
                c = plsc.load_gather(crows[buf], [rows, dv])
                return tuple(
                    accs[k] + c * plsc.load_gather(xrows[buf][k], [rows, dv])
                    for k in range(CTX)
                )

            accs = lax.fori_loop(
                0, EMBED, dbody,
                tuple(jnp.zeros((L,), jnp.float32) for _ in range(CTX)),
            )
            m = accs[0]
            for k in range(1, CTX):
                m = jnp.maximum(m, accs[k])
            es = [jnp.exp(a - m) for a in accs]
            tot = es[0]
            for k in range(1, CTX):
                tot = tot + es[k]
            inv = 1.0 / tot
            orow = rows * CTX
            for k in range(CTX):
                plsc.store_scatter(outv[buf], [orow + k], es[k] * inv)
        base = wid * BPW + g * CHUNK
        pend_out[buf] = pltpu.async_copy(
            outv[buf], out_hbm.at[pl.ds(base * CTX, CHUNK * CTX)], osems[buf])
    for cp in pend_out:
        if cp is not None:
            cp.wait()


def kernel(center, context, center_table, context_table):
    center_r = center.reshape(NW, NCHUNK, CHUNK)
    ctxt_r = context.T.reshape(CTX, NW, NCHUNK, CHUNK)
    out = _w2v(center_r, ctxt_r, center_table, context_table)
    return out.reshape(BATCH, CTX)


# speedup vs baseline: 2.1244x; 1.0293x over previous
import functools

import jax
import jax.numpy as jnp
from jax import lax
from jax.experimental import pallas as pl
from jax.experimental.pallas import tpu as pltpu
from jax.experimental.pallas import tpu_sc as plsc

VOCAB = 100000
EMBED = 128
BATCH = 16384
CTX = 6

NC = 2
NS = 16
L = 16
NW = NC * NS
BPW = BATCH // NW
CHUNK = 64
NCHUNK = BPW // CHUNK
NBUF = 2

_MESH = plsc.VectorSubcoreMesh(
    core_axis_name="c", subcore_axis_name="s", num_cores=NC, num_subcores=NS
)


@functools.partial(
    pl.kernel,
    out_type=jax.ShapeDtypeStruct((BATCH * CTX,), jnp.float32),
    mesh=_MESH,
    scratch_types=[
        pltpu.VMEM((NCHUNK, CHUNK), jnp.int32),
        [pltpu.VMEM((NCHUNK, CHUNK), jnp.int32) for _ in range(CTX)],
        [pltpu.VMEM((CHUNK, EMBED), jnp.float32) for _ in range(NBUF)],
        [[pltpu.VMEM((CHUNK, EMBED), jnp.float32) for _ in range(CTX)]
         for _ in range(NBUF)],
        [pltpu.VMEM((CHUNK * CTX,), jnp.float32) for _ in range(NBUF)],
        [pltpu.SemaphoreType.DMA for _ in range(NBUF)],
        [pltpu.SemaphoreType.DMA for _ in range(NBUF)],
    ],
    compiler_params=pltpu.CompilerParams(needs_layout_passes=False),
)
def _w2v(center_hbm, ctxt_hbm, ctable_hbm, xtable_hbm, out_hbm,
         cidx, xidx, crows, xrows, outv, sems, osems):
    wid = lax.axis_index("s") * NC + lax.axis_index("c")
    stage = [pltpu.async_copy(center_hbm.at[wid, 0], cidx.at[0], osems[0])]
    for k in range(CTX):
        stage.append(pltpu.async_copy(ctxt_hbm.at[k, wid, 0], xidx[k].at[0],
                                      osems[0]))
    for cp in stage:
        cp.wait()

    def fire(g, buf):
        cps = [pltpu.async_copy(ctable_hbm.at[cidx.at[g]], crows[buf],
                                sems[buf])]
        for k in range(CTX):
            cps.append(pltpu.async_copy(xtable_hbm.at[xidx[k].at[g]],
                                        xrows[buf][k], sems[buf]))
        return cps

    pend = fire(0, 0)
    rest = pl.ds(1, NCHUNK - 1)
    stage = [pltpu.async_copy(center_hbm.at[wid, rest], cidx.at[rest],
                              osems[1])]
    for k in range(CTX):
        stage.append(pltpu.async_copy(ctxt_hbm.at[k, wid, rest],
                                      xidx[k].at[rest], osems[1]))
    for cp in stage:
        cp.wait()
    pend_out = [None] * NBUF
    for g in range(NCHUNK):
        buf = g % NBUF
        pend_next = fire(g + 1, (g + 1) % NBUF) if g + 1 < NCHUNK else None
        for cp in pend:
            cp.wait()
        pend = pend_next
        if pend_out[buf] is not None:
            pend_out[buf].wait()
        for sub in range(CHUNK // L):
            lane = lax.iota(jnp.int32, L)
            rows = lane + sub * L

            def dbody(d, accs):
                dv = jnp.bitwise_and(lane + d, EMBED - 1)
                c = plsc.load_gather(crows[buf], [rows, dv])
                return tuple(
                    accs[k] + c * plsc.load_gather(xrows[buf][k], [rows, dv])
                    for k in range(CTX)
                )

            accs = lax.fori_loop(
                0, EMBED, dbody,
                tuple(jnp.zeros((L,), jnp.float32) for _ in range(CTX)),
            )
            m = accs[0]
            for k in range(1, CTX):
                m = jnp.maximum(m, accs[k])
            es = [jnp.exp(a - m) for a in accs]
            tot = es[0]
            for k in range(1, CTX):
                tot = tot + es[k]
            inv = 1.0 / tot
            orow = rows * CTX
            for k in range(CTX):
                plsc.store_scatter(outv[buf], [orow + k], es[k] * inv)
        base = wid * BPW + g * CHUNK
        pend_out[buf] = pltpu.async_copy(
            outv[buf], out_hbm.at[pl.ds(base * CTX, CHUNK * CTX)], osems[buf])
    for cp in pend_out:
        if cp is not None:
            cp.wait()


def kernel(center, context, center_table, context_table):
    center_r = center.reshape(NW, NCHUNK, CHUNK)
    ctxt_r = context.T.reshape(CTX, NW, NCHUNK, CHUNK)
    out = _w2v(center_r, ctxt_r, center_table, context_table)
    return out.reshape(BATCH, CTX)


# speedup vs baseline: 2.1460x; 1.0102x over previous
import functools

import jax
import jax.numpy as jnp
from jax import lax
from jax.experimental import pallas as pl
from jax.experimental.pallas import tpu as pltpu
from jax.experimental.pallas import tpu_sc as plsc

VOCAB = 100000
EMBED = 128
BATCH = 16384
CTX = 6

NC = 2
NS = 16
L = 16
NW = NC * NS
BPW = BATCH // NW
CHUNK = 64
NCHUNK = BPW // CHUNK
NBUF = 2

_MESH = plsc.VectorSubcoreMesh(
    core_axis_name="c", subcore_axis_name="s", num_cores=NC, num_subcores=NS
)


@functools.partial(
    pl.kernel,
    out_type=jax.ShapeDtypeStruct((BATCH * CTX,), jnp.float32),
    mesh=_MESH,
    scratch_types=[
        pltpu.VMEM((NCHUNK, CHUNK), jnp.int32),
        [pltpu.VMEM((NCHUNK, CHUNK), jnp.int32) for _ in range(CTX)],
        [pltpu.VMEM((CHUNK, EMBED), jnp.float32) for _ in range(NBUF)],
        [[pltpu.VMEM((CHUNK, EMBED), jnp.float32) for _ in range(CTX)]
         for _ in range(NBUF)],
        [pltpu.VMEM((CHUNK * CTX,), jnp.float32) for _ in range(NBUF)],
        [pltpu.SemaphoreType.DMA for _ in range(NBUF)],
        [pltpu.SemaphoreType.DMA for _ in range(NBUF)],
    ],
    compiler_params=pltpu.CompilerParams(needs_layout_passes=False),
)
def _w2v(center_hbm, ctxt_hbm, ctable_hbm, xtable_hbm, out_hbm,
         cidx, xidx, crows, xrows, outv, sems, osems):
    wid = lax.axis_index("s") * NC + lax.axis_index("c")
    stage = [pltpu.async_copy(center_hbm.at[wid, 0], cidx.at[0], osems[0])]
    for k in range(CTX):
        stage.append(pltpu.async_copy(ctxt_hbm.at[k, wid, 0], xidx[k].at[0],
                                      osems[0]))
    for cp in stage:
        cp.wait()

    def fire(g, buf):
        cps = [pltpu.async_copy(ctable_hbm.at[cidx.at[g]], crows[buf],
                                sems[buf])]
        for k in range(CTX):
            cps.append(pltpu.async_copy(xtable_hbm.at[xidx[k].at[g]],
                                        xrows[buf][k], sems[buf]))
        return cps

    def fire_half(h, sem):
        half = pl.ds(h * (CHUNK // 2), CHUNK // 2)
        cps = [pltpu.async_copy(ctable_hbm.at[cidx.at[0, half]],
                                crows[0].at[half], sem)]
        for k in range(CTX):
            cps.append(pltpu.async_copy(xtable_hbm.at[xidx[k].at[0, half]],
                                        xrows[0][k].at[half], sem))
        return cps

    pend_half = [fire_half(0, osems[0]), fire_half(1, sems[0])]
    rest = pl.ds(1, NCHUNK - 1)
    stage = [pltpu.async_copy(center_hbm.at[wid, rest], cidx.at[rest],
                              osems[1])]
    for k in range(CTX):
        stage.append(pltpu.async_copy(ctxt_hbm.at[k, wid, rest],
                                      xidx[k].at[rest], osems[1]))
    for cp in stage:
        cp.wait()
    def do_sub(buf, sub):
        lane = lax.iota(jnp.int32, L)
        rows = lane + sub * L

        def dbody(d, accs):
            dv = jnp.bitwise_and(lane + d, EMBED - 1)
            c = plsc.load_gather(crows[buf], [rows, dv])
            return tuple(
                accs[k] + c * plsc.load_gather(xrows[buf][k], [rows, dv])
                for k in range(CTX)
            )

        accs = lax.fori_loop(
            0, EMBED, dbody,
            tuple(jnp.zeros((L,), jnp.float32) for _ in range(CTX)),
        )
        m = accs[0]
        for k in range(1, CTX):
            m = jnp.maximum(m, accs[k])
        es = [jnp.exp(a - m) for a in accs]
        tot = es[0]
        for k in range(1, CTX):
            tot = tot + es[k]
        inv = 1.0 / tot
        orow = rows * CTX
        for k in range(CTX):
            plsc.store_scatter(outv[buf], [orow + k], es[k] * inv)

    def ship_out(g, buf):
        base = wid * BPW + g * CHUNK
        return pltpu.async_copy(
            outv[buf], out_hbm.at[pl.ds(base * CTX, CHUNK * CTX)], osems[buf])

    pend_out = [None] * NBUF
    pend = fire(1, 1)
    for cp in pend_half[0]:
        cp.wait()
    for sub in range(CHUNK // L // 2):
        do_sub(0, sub)
    for cp in pend_half[1]:
        cp.wait()
    for sub in range(CHUNK // L // 2, CHUNK // L):
        do_sub(0, sub)
    pend_out[0] = ship_out(0, 0)
    for g in range(1, NCHUNK):
        buf = g % NBUF
        pend_next = fire(g + 1, (g + 1) % NBUF) if g + 1 < NCHUNK else None
        for cp in pend:
            cp.wait()
        pend = pend_next
        if pend_out[buf] is not None:
            pend_out[buf].wait()
        for sub in range(CHUNK // L):
            do_sub(buf, sub)
        pend_out[buf] = ship_out(g, buf)
    for cp in pend_out:
        if cp is not None:
            cp.wait()


def kernel(center, context, center_table, context_table):
    center_r = center.reshape(NW, NCHUNK, CHUNK)
    ctxt_r = context.T.reshape(CTX, NW, NCHUNK, CHUNK)
    out = _w2v(center_r, ctxt_r, center_table, context_table)
    return out.reshape(BATCH, CTX)
